# Initial kernel scaffold; baseline (speedup 1.0000x reference)
import jax
import jax.numpy as jnp
from jax.experimental import pallas as pl


def kernel(user_emb, item_emb, edge_index):
    raise NotImplementedError("write your pallas kernel here")


# speedup vs baseline: 11.4126x; 11.4126x over previous
import functools

import jax
import jax.numpy as jnp
from jax import lax
from jax.experimental import pallas as pl
from jax.experimental.pallas import tpu as pltpu
from jax.experimental.pallas import tpu_sc as plsc

NU = 25000
NI = 25000
N = NU + NI
D = 64
LAYERS = 3
E = 800000
NC = 2
NS = 16
L = 16
HALF = N // NC
ROWS_T = 1568
P = NS * ROWS_T
NP = NC * P
PAD = P - HALF
DUMMY = HALF + 8
K = 80
EPT = E // NS
CPT = EPT // K
ZR = 112


def _mesh():
    return plsc.VectorSubcoreMesh(
        core_axis_name="c", subcore_axis_name="s",
        num_cores=NC, num_subcores=NS)


_SC_PARAMS = pltpu.CompilerParams(use_tc_tiling_on_sc=False,
                                  needs_layout_passes=False)


NBUF = 5
KG = NBUF * K
NG = CPT // NBUF
RG = ROWS_T // L


@functools.partial(
    pl.kernel,
    out_type=jax.ShapeDtypeStruct((NP,), jnp.float32),
    mesh=_mesh(),
    compiler_params=_SC_PARAMS,
    scratch_types=(
        [pltpu.VMEM((KG,), jnp.int32) for _ in range(2)] +
        [pltpu.VMEM((P,), jnp.float32),
         pltpu.VMEM((NS, ROWS_T), jnp.float32),
         pltpu.VMEM_SHARED((NS, P), jnp.float32)] +
        [pltpu.SemaphoreType.DMA for _ in range(2)]
    ),
)
def _deg_kernel(dst_hbm, zeros_hbm, deg_hbm, *scr):
    gdst = scr[0:2]
    degv = scr[2]
    rbuf = scr[3]
    sacc = scr[4]
    isem = scr[5:7]
    c = lax.axis_index("c")
    s = lax.axis_index("s")
    pltpu.sync_copy(zeros_hbm, degv)
    c0 = c * HALF
    ebase = s * EPT
    one = jnp.full((L,), 1.0, jnp.float32)

    def issue_idx(g, p):
        base = pl.multiple_of(ebase + g * KG, 8)
        pltpu.async_copy(dst_hbm.at[pl.ds(base, KG)], gdst[p], isem[p])

    def wait_idx(p):
        pltpu.make_async_copy(dst_hbm.at[pl.ds(0, KG)], gdst[p], isem[p]).wait()

    def accum_group(p):
        for j in range(KG // L):
            dv = gdst[p][pl.ds(j * L, L)] - c0
            ok = (dv >= 0) & (dv < HALF)
            lidx = jnp.where(ok, dv, DUMMY)
            plsc.addupdate_scatter(degv, [lidx], one, mask=ok)

    issue_idx(0, 0)
    issue_idx(1, 1)
    wait_idx(0)
    accum_group(0)

    def pair(h, _):
        for q in range(2):
            g = 1 + 2 * h + q
            p = (1 + q) % 2
            wait_idx(p)

            @pl.when(g < NG - 1)
            def _():
                issue_idx(g + 1, q)

            accum_group(p)
        return 0

    lax.fori_loop(0, (NG - 1) // 2, pair, 0)
    pltpu.sync_copy(degv, sacc.at[s])
    plsc.subcore_barrier()
    pltpu.sync_copy(sacc.at[:, pl.ds(s * ROWS_T, ROWS_T)], rbuf)

    def red(i, _):
        o = pl.multiple_of(i * L, L)
        v = rbuf[0, pl.ds(o, L)]
        for r in range(1, NS):
            v = v + rbuf[r, pl.ds(o, L)]
        degv[pl.ds(o, L)] = v
        return 0

    lax.fori_loop(0, RG, red, 0)
    pltpu.sync_copy(degv.at[pl.ds(0, ROWS_T)],
                    deg_hbm.at[pl.ds(c * P + s * ROWS_T, ROWS_T)])


_PROP_SCRATCH = (
    [pltpu.VMEM((KG,), jnp.int32) for _ in range(2)] +
    [pltpu.VMEM((KG,), jnp.int32) for _ in range(2)] +
    [pltpu.VMEM((K,), jnp.int32) for _ in range(NBUF)] +
    [pltpu.VMEM((K,), jnp.int32) for _ in range(NBUF)] +
    [pltpu.VMEM((K, D), jnp.float32) for _ in range(NBUF)] +
    [pltpu.VMEM_SHARED((P, D), jnp.float32)] +
    [pltpu.SemaphoreType.DMA for _ in range(2 + 2 * NBUF)]
)


@functools.partial(
    pl.kernel,
    out_type=jax.ShapeDtypeStruct((NP, D), jnp.float32),
    mesh=_mesh(),
    compiler_params=_SC_PARAMS,
    scratch_types=_PROP_SCRATCH,
)
def _prop_kernel(y_hbm, src_hbm, dst_hbm, zeros_hbm, out_hbm, *scr):
    gsrc = scr[0:2]
    gdst = scr[2:4]
    tsrc = scr[4:4 + NBUF]
    tdst = scr[4 + NBUF:4 + 2 * NBUF]
    msg = scr[4 + 2 * NBUF:4 + 3 * NBUF]
    acc = scr[4 + 3 * NBUF]
    isem = scr[5 + 3 * NBUF:7 + 3 * NBUF]
    gsem = scr[7 + 3 * NBUF:7 + 4 * NBUF]
    ssem = scr[7 + 4 * NBUF:]
    c = lax.axis_index("c")
    s = lax.axis_index("s")
    pltpu.sync_copy(zeros_hbm, acc.at[pl.ds(s * ROWS_T, ROWS_T)])
    plsc.subcore_barrier()
    c0 = c * HALF
    ebase = s * EPT

    def issue_idx(g, p):
        base = pl.multiple_of(ebase + g * KG, 8)
        pltpu.async_copy(src_hbm.at[pl.ds(base, KG)], gsrc[p], isem[p])
        pltpu.async_copy(dst_hbm.at[pl.ds(base, KG)], gdst[p], isem[p])

    def wait_idx(p):
        pltpu.make_async_copy(src_hbm.at[pl.ds(0, KG)], gsrc[p], isem[p]).wait()
        pltpu.make_async_copy(dst_hbm.at[pl.ds(0, KG)], gdst[p], isem[p]).wait()

    def transform_and_gather(p, u):
        for j in range(K // L):
            o = u * K + j * L
            sv = gsrc[p][pl.ds(o, L)]
            tsrc[u][pl.ds(j * L, L)] = jnp.where(sv >= HALF, sv + PAD, sv)
            dv = gdst[p][pl.ds(o, L)] - c0
            ok = (dv >= 0) & (dv < HALF)
            tdst[u][pl.ds(j * L, L)] = jnp.where(ok, dv, DUMMY)
        pltpu.async_copy(y_hbm.at[tsrc[u]], msg[u], gsem[u])

    def wait_gather_start_scatter(u):
        pltpu.make_async_copy(y_hbm.at[tsrc[u]], msg[u], gsem[u]).wait()
        pltpu.async_copy(msg[u], acc.at[tdst[u]], ssem[u], add=True)

    def wait_scatter(u):
        pltpu.make_async_copy(msg[u], acc.at[tdst[u]], ssem[u]).wait()

    issue_idx(0, 0)
    issue_idx(1, 1)
    wait_idx(0)
    for u in range(NBUF):
        transform_and_gather(0, u)

    def pair(h, _):
        for q in range(2):
            g = 1 + 2 * h + q
            p = (1 + q) % 2
            for u in range(NBUF):
                wait_gather_start_scatter(u)

            @pl.when(g < NG - 1)
            def _():
                issue_idx(g + 1, q)

            wait_idx(p)
            for u in range(NBUF):
                wait_scatter(u)
                transform_and_gather(p, u)
        return 0

    lax.fori_loop(0, (NG - 1) // 2, pair, 0)
    for u in range(NBUF):
        wait_gather_start_scatter(u)
    for u in range(NBUF):
        wait_scatter(u)
    plsc.subcore_barrier()
    pltpu.sync_copy(acc.at[pl.ds(s * ROWS_T, ROWS_T)],
                    out_hbm.at[pl.ds(c * P + s * ROWS_T, ROWS_T)])


_R = 784


def _tc_spec(cols):
    return pl.BlockSpec((_R, cols), lambda i: (i, 0))


def _tc_dis_scale(degp, x0p):
    def body(deg_ref, x_ref, dis_ref, y_ref):
        deg = deg_ref[...]
        dis = jnp.where(deg > 0, lax.rsqrt(deg), 0.0)
        dis_ref[...] = dis
        y_ref[...] = x_ref[...] * dis
    return pl.pallas_call(
        body,
        grid=(NP // _R,),
        in_specs=[_tc_spec(1), _tc_spec(D)],
        out_specs=[_tc_spec(1), _tc_spec(D)],
        out_shape=[jax.ShapeDtypeStruct((NP, 1), jnp.float32),
                   jax.ShapeDtypeStruct((NP, D), jnp.float32)],
    )(degp, x0p)


def _tc_rescale(acc, dis):
    def body(a_ref, d_ref, y_ref):
        d = d_ref[...]
        y_ref[...] = a_ref[...] * (d * d)
    return pl.pallas_call(
        body,
        grid=(NP // _R,),
        in_specs=[_tc_spec(D), _tc_spec(1)],
        out_specs=_tc_spec(D),
        out_shape=jax.ShapeDtypeStruct((NP, D), jnp.float32),
    )(acc, dis)


def _tc_mean(x0p, dis, a0, a1, a2):
    def body(x_ref, d_ref, a0_ref, a1_ref, a2_ref, o_ref):
        acc = a0_ref[...] + a1_ref[...] + a2_ref[...]
        o_ref[...] = 0.25 * (x_ref[...] + d_ref[...] * acc)
    return pl.pallas_call(
        body,
        grid=(NP // _R,),
        in_specs=[_tc_spec(D), _tc_spec(1), _tc_spec(D), _tc_spec(D), _tc_spec(D)],
        out_specs=_tc_spec(D),
        out_shape=jax.ShapeDtypeStruct((NP, D), jnp.float32),
    )(x0p, dis, a0, a1, a2)


def kernel(user_emb, item_emb, edge_index):
    src = edge_index[0]
    dst = edge_index[1]
    zpad = jnp.zeros((PAD, D), jnp.float32)
    x0p = jnp.concatenate([user_emb, zpad, item_emb, zpad], axis=0)
    zeros_half = jnp.zeros((P,), jnp.float32)
    zerosr = jnp.zeros((ROWS_T, D), jnp.float32)

    degp = _deg_kernel(dst, zeros_half).reshape(NP, 1)
    dis, y = _tc_dis_scale(degp, x0p)
    accs = []
    for l in range(LAYERS):
        acc = _prop_kernel(y, src, dst, zerosr)
        accs.append(acc)
        if l < LAYERS - 1:
            y = _tc_rescale(acc, dis)
    out = _tc_mean(x0p, dis, accs[0], accs[1], accs[2])
    return out[:NU], out[P:P + NI]
